# baseline (speedup 1.0000x reference)
import functools

import jax
import jax.numpy as jnp
from jax import lax
from jax.experimental import pallas as pl
from jax.experimental.pallas import tpu as pltpu
from jax.experimental.pallas import tpu_sc as plsc

N = 50000
E = 800000
T = 4
H = 64

NPAD = 50176
EPAD = 819200
ROWS_E = EPAD // 128
RPT32 = ROWS_E // 32
RPT16 = ROWS_E // 16
NSL = NPAD // 16
RB = 1024
GRID = NPAD // RB

_MESH = dict(core_axis_name="c", subcore_axis_name="s", num_cores=2,
             num_subcores=16)
_SC_PARAMS = pltpu.CompilerParams(use_tc_tiling_on_sc=False)


def _edge_pipeline(nb, n_mid, fire_idx, wait_idx, fire_g, wait_g, fire_s,
                   drain_s):
    fire_idx(0, 0)
    fire_idx(1, 1)
    fire_idx(2, 2)
    wait_idx(0)
    fire_g(0, 0, 0)
    wait_idx(1)
    fire_g(1, 1, 1)

    def step(i, k, do_fidx=True, do_fg=True, do_drain=True):
        if do_fidx:
            fire_idx(i + 3, (k + 3) % 8)
        if do_drain:
            drain_s((k + 2) % 4)
        if do_fg:
            wait_idx((k + 2) % 8)
            fire_g(i + 2, (k + 2) % 8, (k + 2) % 4)
        wait_g(k % 4)
        fire_s(k, k % 4)

    for k in range(8):
        step(k, k, do_drain=(k >= 2))

    def mid(m, carry):
        base = 8 + 8 * m
        for k in range(8):
            step(base + k, k)
        return carry

    lax.fori_loop(0, n_mid, mid, 0)
    b0 = nb - 8
    for k in range(8):
        step(b0 + k, k, do_fidx=(k < 5), do_fg=(k < 6))
    drain_s(2)
    drain_s(3)


def _hist_body(dst_hbm, zdeg_hbm, ones_hbm, degp_hbm,
               didx, ones_v, vbuf, acc0, acc1, acc2, acc3):
    accs = (acc0, acc1, acc2, acc3)
    c = lax.axis_index("c")
    s = lax.axis_index("s")
    w = c * 16 + s
    row0 = w * RPT32
    pltpu.sync_copy(ones_hbm, ones_v)
    pltpu.sync_copy(zdeg_hbm.at[pl.ds(s * NSL, NSL)], vbuf)
    for t in range(T):
        pltpu.sync_copy(vbuf, accs[t].at[pl.ds(s * NSL, NSL)])
    plsc.subcore_barrier()

    def body(i, carry):
        for t in range(T):
            pltpu.sync_copy(dst_hbm.at[t, pl.ds(row0 + i * 4, 4)], didx)
            for j in range(4):
                pltpu.sync_copy(ones_v, accs[t].at[didx.at[j]], add=True)
        return carry

    lax.fori_loop(0, RPT32 // 4, body, 0)
    plsc.subcore_barrier()
    for t in range(T):
        off = pl.multiple_of((c * T + t) * NPAD + s * NSL, 128)
        pltpu.sync_copy(accs[t].at[pl.ds(s * NSL, NSL)], vbuf)
        pltpu.sync_copy(vbuf, degp_hbm.at[pl.ds(off, NSL)])


def _make_pass0_body(ts):
    nt = len(ts)

    def body(y0_hbm, srcf_hbm, dstf_hbm, z16_hbm, z0p_hbm,
             sidx, didx, rows,
             i0, i1, i2, i3, i4, i5, i6, i7,
             g0, g1, g2, g3, s0, s1, s2, s3,
             acc):
        isems = (i0, i1, i2, i3, i4, i5, i6, i7)
        gsems = (g0, g1, g2, g3)
        ssems = (s0, s1, s2, s3)
        c = lax.axis_index("c")
        s_ = lax.axis_index("s")
        w = c * 16 + s_
        row0 = w * RPT32
        for ti, t in enumerate(ts):
            pltpu.sync_copy(z16_hbm.at[pl.ds(s_ * NSL, NSL), :],
                            acc.at[pl.ds(s_ * NSL, NSL), :])
            plsc.subcore_barrier()

            def fire_idx(i, b8, t=t):
                off = pl.multiple_of((t * ROWS_E + row0 + i) * 128, 128)
                pltpu.async_copy(srcf_hbm.at[pl.ds(off, 128)], sidx.at[b8],
                                 isems[b8])
                pltpu.async_copy(dstf_hbm.at[pl.ds(off, 128)], didx.at[b8],
                                 isems[b8])

            def wait_idx(b8):
                pltpu.make_async_copy(srcf_hbm.at[pl.ds(0, 128)],
                                      sidx.at[b8], isems[b8]).wait()
                pltpu.make_async_copy(srcf_hbm.at[pl.ds(0, 128)],
                                      didx.at[b8], isems[b8]).wait()

            def fire_g(i, b8, b4):
                pltpu.async_copy(y0_hbm.at[sidx.at[b8]], rows.at[b4],
                                 gsems[b4])

            def wait_g(b4):
                pltpu.make_async_copy(y0_hbm.at[pl.ds(0, 128)], rows.at[b4],
                                      gsems[b4]).wait()

            def fire_s(b8, b4):
                pltpu.async_copy(rows.at[b4], acc.at[didx.at[b8]], ssems[b4],
                                 add=True)

            def drain_s(b4):
                pltpu.make_async_copy(rows.at[b4], acc.at[pl.ds(0, 128), :],
                                      ssems[b4]).wait()

            _edge_pipeline(RPT32, (RPT32 - 16) // 8, fire_idx, wait_idx,
                           fire_g, wait_g, fire_s, drain_s)
            plsc.subcore_barrier()
            off = pl.multiple_of((c * nt + ti) * NPAD + s_ * NSL, 128)
            pltpu.sync_copy(acc.at[pl.ds(s_ * NSL, NSL), :],
                            z0p_hbm.at[pl.ds(off, NSL), :])
            plsc.subcore_barrier()

    return body


def _make_pass1_body(ts):
    nt = len(ts)

    def body(*refs):
        ytabs = refs[:2 * nt]
        srcf_hbm, dstf_hbm, z32_hbm, z1p_hbm = refs[2 * nt:2 * nt + 4]
        sidx, didx, rows = refs[2 * nt + 4:2 * nt + 7]
        isems = refs[2 * nt + 7:2 * nt + 15]
        gsems = refs[2 * nt + 15:2 * nt + 19]
        ssems = refs[2 * nt + 19:2 * nt + 23]
        acc = refs[2 * nt + 23]
        ys = tuple(ytabs[2 * i:2 * i + 2] for i in range(nt))
        c = lax.axis_index("c")
        s_ = lax.axis_index("s")
        row0 = s_ * RPT16
        for ti, t in enumerate(ts):
            pltpu.sync_copy(z32_hbm.at[pl.ds(s_ * NSL, NSL), :],
                            acc.at[pl.ds(s_ * NSL, NSL), :])
            plsc.subcore_barrier()
            ytab = ys[ti]

            def fire_idx(i, b8, t=t):
                off = pl.multiple_of((t * ROWS_E + row0 + i) * 128, 128)
                pltpu.async_copy(srcf_hbm.at[pl.ds(off, 128)], sidx.at[b8],
                                 isems[b8])
                pltpu.async_copy(dstf_hbm.at[pl.ds(off, 128)], didx.at[b8],
                                 isems[b8])

            def wait_idx(b8):
                pltpu.make_async_copy(srcf_hbm.at[pl.ds(0, 128)],
                                      sidx.at[b8], isems[b8]).wait()
                pltpu.make_async_copy(srcf_hbm.at[pl.ds(0, 128)],
                                      didx.at[b8], isems[b8]).wait()

            def fire_g(i, b8, b4, ytab=ytab):
                for cc in range(2):
                    @pl.when(c == cc)
                    def _():
                        pltpu.async_copy(ytab[cc].at[sidx.at[b8]],
                                         rows.at[b4], gsems[b4])

            def wait_g(b4, ytab=ytab):
                pltpu.make_async_copy(ytab[0].at[pl.ds(0, 128)], rows.at[b4],
                                      gsems[b4]).wait()

            def fire_s(b8, b4):
                pltpu.async_copy(rows.at[b4], acc.at[didx.at[b8]], ssems[b4],
                                 add=True)

            def drain_s(b4):
                pltpu.make_async_copy(rows.at[b4], acc.at[pl.ds(0, 128), :],
                                      ssems[b4]).wait()

            _edge_pipeline(RPT16, (RPT16 - 16) // 8, fire_idx, wait_idx,
                           fire_g, wait_g, fire_s, drain_s)
            plsc.subcore_barrier()
            off = pl.multiple_of((c * nt + ti) * NPAD + s_ * NSL, 128)
            pltpu.sync_copy(acc.at[pl.ds(s_ * NSL, NSL), :],
                            z1p_hbm.at[pl.ds(off, NSL), :])
            plsc.subcore_barrier()

    return body


def _sc_hist(dst3, zdeg, ones128):
    return pl.kernel(
        _hist_body,
        out_type=jax.ShapeDtypeStruct((2 * T * NPAD,), jnp.float32),
        mesh=plsc.VectorSubcoreMesh(**_MESH),
        compiler_params=_SC_PARAMS,
        scratch_types=[
            pltpu.VMEM((4, 128), jnp.int32),
            pltpu.VMEM((128,), jnp.float32),
            pltpu.VMEM((NSL,), jnp.float32),
            pltpu.VMEM_SHARED((NPAD,), jnp.float32),
            pltpu.VMEM_SHARED((NPAD,), jnp.float32),
            pltpu.VMEM_SHARED((NPAD,), jnp.float32),
            pltpu.VMEM_SHARED((NPAD,), jnp.float32),
        ],
    )(dst3, zdeg, ones128)


def _sc_pass0(y0, srcf, dstf, z16, ts):
    nt = len(ts)
    return pl.kernel(
        _make_pass0_body(ts),
        out_type=jax.ShapeDtypeStruct((2 * nt * NPAD, 16), jnp.float32),
        mesh=plsc.VectorSubcoreMesh(**_MESH),
        compiler_params=_SC_PARAMS,
        scratch_types=[
            pltpu.VMEM((8, 128), jnp.int32),
            pltpu.VMEM((8, 128), jnp.int32),
            pltpu.VMEM((4, 128, 16), jnp.float32),
        ] + [pltpu.SemaphoreType.DMA] * 16 + [
            pltpu.VMEM_SHARED((NPAD, 16), jnp.float32),
        ],
    )(y0, srcf, dstf, z16)


def _sc_pass1(ys, srcf, dstf, z32, ts):
    nt = len(ts)
    return pl.kernel(
        _make_pass1_body(ts),
        out_type=jax.ShapeDtypeStruct((2 * nt * NPAD, 32), jnp.float32),
        mesh=plsc.VectorSubcoreMesh(**_MESH),
        compiler_params=_SC_PARAMS,
        scratch_types=[
            pltpu.VMEM((8, 128), jnp.int32),
            pltpu.VMEM((8, 128), jnp.int32),
            pltpu.VMEM((4, 128, 32), jnp.float32),
        ] + [pltpu.SemaphoreType.DMA] * 16 + [
            pltpu.VMEM_SHARED((NPAD, 32), jnp.float32),
        ],
    )(*ys, srcf, dstf, z32)


def _prep_block(degp_ref, xcat_ref, y0_ref, dinv_ref):
    deg = degp_ref[0] + degp_ref[1] + 1.0
    dinv = lax.rsqrt(deg)
    dinv_ref[...] = dinv
    parts = []
    for t in range(T):
        d = dinv[t][:, None]
        parts.append(xcat_ref[:, 3 * t:3 * t + 3] * d)
    parts.append(jnp.zeros((RB, 4), jnp.float32))
    y0_ref[...] = jnp.concatenate(parts, axis=1)


def _make_dense1_block(ts):
    def body(z0p_ref, xcat_ref, dinv_ref, W0_ref, b0_ref, *y1_refs):
        for ti, t in enumerate(ts):
            u = z0p_ref[0, ti] + z0p_ref[1, ti]
            d = dinv_ref[t][:, None]
            a = (u[:, 3 * t:3 * t + 3] * d
                 + xcat_ref[:, 3 * t:3 * t + 3] * (d * d))
            h1 = jnp.maximum(
                jnp.dot(a, W0_ref[...], preferred_element_type=jnp.float32)
                + b0_ref[...], 0.0)
            y1t = h1 * d
            y1_refs[2 * ti][...] = y1t[:, :32]
            y1_refs[2 * ti + 1][...] = y1t[:, 32:]

    return body


def _make_gru_block(ts, first, last):
    def body(*refs):
        z1p_ref = refs[0]
        ytabs = refs[1:1 + 2 * len(ts)]
        k = 1 + 2 * len(ts)
        if not first:
            h_ref = refs[k]
            k += 1
        (dinv_ref, W1_ref, b1_ref, Wih_ref, Whh_ref, bih_ref,
         bhh_ref) = refs[k:k + 7]
        k += 7
        if last:
            Wm1_ref, bm1_ref, Wm2_ref, bm2_ref = refs[k:k + 4]
            k += 4
        out_ref = refs[k]
        h = (jnp.zeros((RB, H), jnp.float32) if first else h_ref[...])
        for ti, t in enumerate(ts):
            z1t = jnp.concatenate([z1p_ref[0, ti], z1p_ref[1, ti]], axis=1)
            y1t = jnp.concatenate([ytabs[2 * ti][...], ytabs[2 * ti + 1][...]],
                                  axis=1)
            d = dinv_ref[t][:, None]
            emb = jnp.maximum(
                jnp.dot((z1t + y1t) * d, W1_ref[...],
                        preferred_element_type=jnp.float32) + b1_ref[...],
                0.0)
            gi = jnp.dot(emb, Wih_ref[...].T,
                         preferred_element_type=jnp.float32) + bih_ref[...]
            gh = jnp.dot(h, Whh_ref[...].T,
                         preferred_element_type=jnp.float32) + bhh_ref[...]
            r = jax.nn.sigmoid(gi[:, :H] + gh[:, :H])
            z = jax.nn.sigmoid(gi[:, H:2 * H] + gh[:, H:2 * H])
            ng = jnp.tanh(gi[:, 2 * H:] + r * gh[:, 2 * H:])
            h = (1.0 - z) * ng + z * h
        if last:
            h1 = jnp.maximum(
                jnp.dot(h, Wm1_ref[...], preferred_element_type=jnp.float32)
                + bm1_ref[...], 0.0)
            out_ref[...] = jax.nn.sigmoid(
                jnp.dot(h1, Wm2_ref[...], preferred_element_type=jnp.float32)
                + bm2_ref[...])
        else:
            out_ref[...] = h

    return body


def _tc_prep(degp, xcat):
    return pl.pallas_call(
        _prep_block,
        grid=(GRID,),
        in_specs=[pl.BlockSpec((2, T, RB), lambda i: (0, 0, i)),
                  pl.BlockSpec((RB, 12), lambda i: (i, 0))],
        out_specs=[pl.BlockSpec((RB, 16), lambda i: (i, 0)),
                   pl.BlockSpec((T, RB), lambda i: (0, i))],
        out_shape=[jax.ShapeDtypeStruct((NPAD, 16), jnp.float32),
                   jax.ShapeDtypeStruct((T, NPAD), jnp.float32)],
    )(degp, xcat)


def _tc_dense1(z0p, xcat, dinv4, W0, b0, ts):
    nt = len(ts)
    return pl.pallas_call(
        _make_dense1_block(ts),
        grid=(GRID,),
        in_specs=[pl.BlockSpec((2, nt, RB, 16), lambda i: (0, 0, i, 0)),
                  pl.BlockSpec((RB, 12), lambda i: (i, 0)),
                  pl.BlockSpec((T, RB), lambda i: (0, i)),
                  pl.BlockSpec((3, H), lambda i: (0, 0)),
                  pl.BlockSpec((H,), lambda i: (0,))],
        out_specs=[pl.BlockSpec((RB, 32), lambda i: (i, 0))] * (2 * nt),
        out_shape=[jax.ShapeDtypeStruct((NPAD, 32), jnp.float32)] * (2 * nt),
    )(z0p, xcat, dinv4, W0, b0)


def _tc_gru_t(t, z1p, ys, h, dinv4, W1, b1, Wih, Whh, bih, bhh,
              Wm1, bm1, Wm2, bm2):
    first = t == 0
    last = t == T - 1
    yspec = [pl.BlockSpec((RB, 32), lambda i: (i, 0))] * 2
    hspec = [] if first else [pl.BlockSpec((RB, H), lambda i: (i, 0))]
    harg = [] if first else [h]
    wspec = [pl.BlockSpec((T, RB), lambda i: (0, i)),
             pl.BlockSpec((H, H), lambda i: (0, 0)),
             pl.BlockSpec((H,), lambda i: (0,)),
             pl.BlockSpec((3 * H, H), lambda i: (0, 0)),
             pl.BlockSpec((3 * H, H), lambda i: (0, 0)),
             pl.BlockSpec((3 * H,), lambda i: (0,)),
             pl.BlockSpec((3 * H,), lambda i: (0,))]
    wargs = [dinv4, W1, b1, Wih, Whh, bih, bhh]
    if last:
        wspec += [pl.BlockSpec((H, H), lambda i: (0, 0)),
                  pl.BlockSpec((H,), lambda i: (0,)),
                  pl.BlockSpec((H, 1), lambda i: (0, 0)),
                  pl.BlockSpec((1,), lambda i: (0,))]
        wargs += [Wm1, bm1, Wm2, bm2]
        out_w = 1
    else:
        out_w = H
    return pl.pallas_call(
        _make_gru_block((t,), first, last),
        grid=(GRID,),
        in_specs=[pl.BlockSpec((2, 1, RB, 32), lambda i: (0, 0, i, 0))]
        + yspec + hspec + wspec,
        out_specs=pl.BlockSpec((RB, out_w), lambda i: (i, 0)),
        out_shape=jax.ShapeDtypeStruct((NPAD, out_w), jnp.float32),
    )(z1p, *ys, *harg, *wargs)


def kernel(node_features_0, node_features_1, node_features_2, node_features_3,
           edges_0, edges_1, edges_2, edges_3,
           W_gcn0, b_gcn0, W_gcn1, b_gcn1, W_ih, W_hh, b_ih, b_hh,
           W_mlp1, b_mlp1, W_mlp2, b_mlp2):
    feats = [node_features_0, node_features_1, node_features_2, node_features_3]
    edges = [edges_0, edges_1, edges_2, edges_3]

    xcat = jnp.concatenate(
        [jnp.pad(x, ((0, NPAD - N), (0, 0))) for x in feats], axis=1)

    pad_src = jnp.arange(E, EPAD, dtype=jnp.int32) % NPAD
    pad_dst = N + jnp.arange(E, EPAD, dtype=jnp.int32) % (NPAD - N)
    srcs, dsts = [], []
    for e in edges:
        srcs.append(jnp.concatenate([e[0], pad_src]))
        dsts.append(jnp.concatenate([e[1], pad_dst]))
    srcf = jnp.stack(srcs).reshape(-1)
    dstf = jnp.stack(dsts).reshape(-1)
    dst3 = dstf.reshape(T, ROWS_E, 128)

    zdeg = jnp.zeros((NPAD,), jnp.float32)
    ones128 = jnp.ones((128,), jnp.float32)
    z16 = jnp.zeros((NPAD, 16), jnp.float32)
    z32 = jnp.zeros((NPAD, 32), jnp.float32)

    degp = _sc_hist(dst3, zdeg, ones128).reshape(2, T, NPAD)
    y0, dinv4 = _tc_prep(degp, xcat)
    z0a = _sc_pass0(y0, srcf, dstf, z16, (0, 1)).reshape(2, 2, NPAD, 16)
    z0b = _sc_pass0(y0, srcf, dstf, z16, (2, 3)).reshape(2, 2, NPAD, 16)
    ysa = _tc_dense1(z0a, xcat, dinv4, W_gcn0, b_gcn0, (0, 1))
    ysb = _tc_dense1(z0b, xcat, dinv4, W_gcn0, b_gcn0, (2, 3))
    ysall = list(ysa) + list(ysb)
    h = None
    for t in range(T):
        ypair = ysall[2 * t:2 * t + 2]
        z1t = _sc_pass1(ypair, srcf, dstf, z32, (t,)).reshape(2, 1, NPAD, 32)
        h = _tc_gru_t(t, z1t, ypair, h, dinv4, W_gcn1, b_gcn1, W_ih, W_hh,
                      b_ih, b_hh, W_mlp1, b_mlp1, W_mlp2, b_mlp2)
    return h[:N]


# speedup vs baseline: 1.0364x; 1.0364x over previous
import functools

import jax
import jax.numpy as jnp
from jax import lax
from jax.experimental import pallas as pl
from jax.experimental.pallas import tpu as pltpu
from jax.experimental.pallas import tpu_sc as plsc

N = 50000
E = 800000
T = 4
H = 64

NPAD = 50176
EPAD = 819200
ROWS_E = EPAD // 128
RPT32 = ROWS_E // 32
RPT16 = ROWS_E // 16
NSL = NPAD // 16
RB = 1024
GRID = NPAD // RB

_MESH = dict(core_axis_name="c", subcore_axis_name="s", num_cores=2,
             num_subcores=16)
_SC_PARAMS = pltpu.CompilerParams(use_tc_tiling_on_sc=False)


def _edge_pipeline(nb, n_mid, fire_idx, wait_idx, fire_g, wait_g, fire_s,
                   drain_s):
    fire_idx(0, 0)
    fire_idx(1, 1)
    fire_idx(2, 2)
    wait_idx(0)
    fire_g(0, 0, 0)
    wait_idx(1)
    fire_g(1, 1, 1)

    def step(i, k, do_fidx=True, do_fg=True, do_drain=True):
        if do_fidx:
            fire_idx(i + 3, (k + 3) % 8)
        if do_drain:
            drain_s((k + 2) % 4)
        if do_fg:
            wait_idx((k + 2) % 8)
            fire_g(i + 2, (k + 2) % 8, (k + 2) % 4)
        wait_g(k % 4)
        fire_s(k, k % 4)

    for k in range(8):
        step(k, k, do_drain=(k >= 2))

    def mid(m, carry):
        base = 8 + 8 * m
        for k in range(8):
            step(base + k, k)
        return carry

    lax.fori_loop(0, n_mid, mid, 0)
    b0 = nb - 8
    for k in range(8):
        step(b0 + k, k, do_fidx=(k < 5), do_fg=(k < 6))
    drain_s(2)
    drain_s(3)


def _hist_body(dst_hbm, zdeg_hbm, ones_hbm, degp_hbm,
               didx, ones_v, vbuf, acc0, acc1, acc2, acc3):
    accs = (acc0, acc1, acc2, acc3)
    c = lax.axis_index("c")
    s = lax.axis_index("s")
    w = c * 16 + s
    row0 = w * RPT32
    pltpu.sync_copy(ones_hbm, ones_v)
    pltpu.sync_copy(zdeg_hbm.at[pl.ds(s * NSL, NSL)], vbuf)
    for t in range(T):
        pltpu.sync_copy(vbuf, accs[t].at[pl.ds(s * NSL, NSL)])
    plsc.subcore_barrier()

    def body(i, carry):
        for t in range(T):
            pltpu.sync_copy(dst_hbm.at[t, pl.ds(row0 + i * 4, 4)], didx)
            for j in range(4):
                pltpu.sync_copy(ones_v, accs[t].at[didx.at[j]], add=True)
        return carry

    lax.fori_loop(0, RPT32 // 4, body, 0)
    plsc.subcore_barrier()
    for t in range(T):
        off = pl.multiple_of((c * T + t) * NPAD + s * NSL, 128)
        pltpu.sync_copy(accs[t].at[pl.ds(s * NSL, NSL)], vbuf)
        pltpu.sync_copy(vbuf, degp_hbm.at[pl.ds(off, NSL)])


def _make_pass0_body(ts):
    nt = len(ts)

    def body(y0_hbm, srcf_hbm, dstf_hbm, z16_hbm, z0p_hbm,
             sidx, didx, rows,
             i0, i1, i2, i3, i4, i5, i6, i7,
             g0, g1, g2, g3, s0, s1, s2, s3,
             acc):
        isems = (i0, i1, i2, i3, i4, i5, i6, i7)
        gsems = (g0, g1, g2, g3)
        ssems = (s0, s1, s2, s3)
        c = lax.axis_index("c")
        s_ = lax.axis_index("s")
        w = c * 16 + s_
        row0 = w * RPT32
        for ti, t in enumerate(ts):
            pltpu.sync_copy(z16_hbm.at[pl.ds(s_ * NSL, NSL), :],
                            acc.at[pl.ds(s_ * NSL, NSL), :])
            plsc.subcore_barrier()

            def fire_idx(i, b8, t=t):
                off = pl.multiple_of((t * ROWS_E + row0 + i) * 128, 128)
                pltpu.async_copy(srcf_hbm.at[pl.ds(off, 128)], sidx.at[b8],
                                 isems[b8])
                pltpu.async_copy(dstf_hbm.at[pl.ds(off, 128)], didx.at[b8],
                                 isems[b8])

            def wait_idx(b8):
                pltpu.make_async_copy(srcf_hbm.at[pl.ds(0, 128)],
                                      sidx.at[b8], isems[b8]).wait()
                pltpu.make_async_copy(srcf_hbm.at[pl.ds(0, 128)],
                                      didx.at[b8], isems[b8]).wait()

            def fire_g(i, b8, b4):
                pltpu.async_copy(y0_hbm.at[sidx.at[b8]], rows.at[b4],
                                 gsems[b4])

            def wait_g(b4):
                pltpu.make_async_copy(y0_hbm.at[pl.ds(0, 128)], rows.at[b4],
                                      gsems[b4]).wait()

            def fire_s(b8, b4):
                pltpu.async_copy(rows.at[b4], acc.at[didx.at[b8]], ssems[b4],
                                 add=True)

            def drain_s(b4):
                pltpu.make_async_copy(rows.at[b4], acc.at[pl.ds(0, 128), :],
                                      ssems[b4]).wait()

            _edge_pipeline(RPT32, (RPT32 - 16) // 8, fire_idx, wait_idx,
                           fire_g, wait_g, fire_s, drain_s)
            plsc.subcore_barrier()
            off = pl.multiple_of((c * nt + ti) * NPAD + s_ * NSL, 128)
            pltpu.sync_copy(acc.at[pl.ds(s_ * NSL, NSL), :],
                            z0p_hbm.at[pl.ds(off, NSL), :])
            plsc.subcore_barrier()

    return body


def _make_pass1_body(ts):
    nt = len(ts)

    def body(*refs):
        ytabs = refs[:2 * nt]
        srcf_hbm, dstf_hbm, z32_hbm, z1p_hbm = refs[2 * nt:2 * nt + 4]
        sidx, didx, rows = refs[2 * nt + 4:2 * nt + 7]
        isems = refs[2 * nt + 7:2 * nt + 15]
        gsems = refs[2 * nt + 15:2 * nt + 19]
        ssems = refs[2 * nt + 19:2 * nt + 23]
        acc = refs[2 * nt + 23]
        ys = tuple(ytabs[2 * i:2 * i + 2] for i in range(nt))
        c = lax.axis_index("c")
        s_ = lax.axis_index("s")
        row0 = s_ * RPT16
        for ti, t in enumerate(ts):
            pltpu.sync_copy(z32_hbm.at[pl.ds(s_ * NSL, NSL), :],
                            acc.at[pl.ds(s_ * NSL, NSL), :])
            plsc.subcore_barrier()
            ytab = ys[ti]

            def fire_idx(i, b8, t=t):
                off = pl.multiple_of((t * ROWS_E + row0 + i) * 128, 128)
                pltpu.async_copy(srcf_hbm.at[pl.ds(off, 128)], sidx.at[b8],
                                 isems[b8])
                pltpu.async_copy(dstf_hbm.at[pl.ds(off, 128)], didx.at[b8],
                                 isems[b8])

            def wait_idx(b8):
                pltpu.make_async_copy(srcf_hbm.at[pl.ds(0, 128)],
                                      sidx.at[b8], isems[b8]).wait()
                pltpu.make_async_copy(srcf_hbm.at[pl.ds(0, 128)],
                                      didx.at[b8], isems[b8]).wait()

            def fire_g(i, b8, b4, ytab=ytab):
                for cc in range(2):
                    @pl.when(c == cc)
                    def _():
                        pltpu.async_copy(ytab[cc].at[sidx.at[b8]],
                                         rows.at[b4], gsems[b4])

            def wait_g(b4, ytab=ytab):
                pltpu.make_async_copy(ytab[0].at[pl.ds(0, 128)], rows.at[b4],
                                      gsems[b4]).wait()

            def fire_s(b8, b4):
                pltpu.async_copy(rows.at[b4], acc.at[didx.at[b8]], ssems[b4],
                                 add=True)

            def drain_s(b4):
                pltpu.make_async_copy(rows.at[b4], acc.at[pl.ds(0, 128), :],
                                      ssems[b4]).wait()

            _edge_pipeline(RPT16, (RPT16 - 16) // 8, fire_idx, wait_idx,
                           fire_g, wait_g, fire_s, drain_s)
            plsc.subcore_barrier()
            off = pl.multiple_of((c * nt + ti) * NPAD + s_ * NSL, 128)
            pltpu.sync_copy(acc.at[pl.ds(s_ * NSL, NSL), :],
                            z1p_hbm.at[pl.ds(off, NSL), :])
            plsc.subcore_barrier()

    return body


def _sc_hist(dst3, zdeg, ones128):
    return pl.kernel(
        _hist_body,
        out_type=jax.ShapeDtypeStruct((2 * T * NPAD,), jnp.float32),
        mesh=plsc.VectorSubcoreMesh(**_MESH),
        compiler_params=_SC_PARAMS,
        scratch_types=[
            pltpu.VMEM((4, 128), jnp.int32),
            pltpu.VMEM((128,), jnp.float32),
            pltpu.VMEM((NSL,), jnp.float32),
            pltpu.VMEM_SHARED((NPAD,), jnp.float32),
            pltpu.VMEM_SHARED((NPAD,), jnp.float32),
            pltpu.VMEM_SHARED((NPAD,), jnp.float32),
            pltpu.VMEM_SHARED((NPAD,), jnp.float32),
        ],
    )(dst3, zdeg, ones128)


def _sc_pass0(y0, srcf, dstf, z16, ts):
    nt = len(ts)
    return pl.kernel(
        _make_pass0_body(ts),
        out_type=jax.ShapeDtypeStruct((2 * nt * NPAD, 16), jnp.float32),
        mesh=plsc.VectorSubcoreMesh(**_MESH),
        compiler_params=_SC_PARAMS,
        scratch_types=[
            pltpu.VMEM((8, 128), jnp.int32),
            pltpu.VMEM((8, 128), jnp.int32),
            pltpu.VMEM((4, 128, 16), jnp.float32),
        ] + [pltpu.SemaphoreType.DMA] * 16 + [
            pltpu.VMEM_SHARED((NPAD, 16), jnp.float32),
        ],
    )(y0, srcf, dstf, z16)


def _sc_pass1(ys, srcf, dstf, z32, ts):
    nt = len(ts)
    return pl.kernel(
        _make_pass1_body(ts),
        out_type=jax.ShapeDtypeStruct((2 * nt * NPAD, 32), jnp.float32),
        mesh=plsc.VectorSubcoreMesh(**_MESH),
        compiler_params=_SC_PARAMS,
        scratch_types=[
            pltpu.VMEM((8, 128), jnp.int32),
            pltpu.VMEM((8, 128), jnp.int32),
            pltpu.VMEM((4, 128, 32), jnp.float32),
        ] + [pltpu.SemaphoreType.DMA] * 16 + [
            pltpu.VMEM_SHARED((NPAD, 32), jnp.float32),
        ],
    )(*ys, srcf, dstf, z32)


def _prep_block(degp_ref, xcat_ref, y0_ref, dinv_ref):
    deg = degp_ref[0] + degp_ref[1] + 1.0
    dinv = lax.rsqrt(deg)
    dinv_ref[...] = dinv
    parts = []
    for t in range(T):
        d = dinv[t][:, None]
        parts.append(xcat_ref[:, 3 * t:3 * t + 3] * d)
    parts.append(jnp.zeros((RB, 4), jnp.float32))
    y0_ref[...] = jnp.concatenate(parts, axis=1)


def _make_dense1_block(ts):
    def body(z0p_ref, xcat_ref, dinv_ref, W0_ref, b0_ref, *y1_refs):
        for ti, t in enumerate(ts):
            u = z0p_ref[0, ti] + z0p_ref[1, ti]
            d = dinv_ref[t][:, None]
            a = (u[:, 3 * t:3 * t + 3] * d
                 + xcat_ref[:, 3 * t:3 * t + 3] * (d * d))
            h1 = jnp.maximum(
                jnp.dot(a, W0_ref[...], preferred_element_type=jnp.float32)
                + b0_ref[...], 0.0)
            y1t = h1 * d
            y1_refs[2 * ti][...] = y1t[:, :32]
            y1_refs[2 * ti + 1][...] = y1t[:, 32:]

    return body


def _make_gru_block(ts, first, last):
    def body(*refs):
        z1p_ref = refs[0]
        ytabs = refs[1:1 + 2 * len(ts)]
        k = 1 + 2 * len(ts)
        if not first:
            h_ref = refs[k]
            k += 1
        (dinv_ref, W1_ref, b1_ref, Wih_ref, Whh_ref, bih_ref,
         bhh_ref) = refs[k:k + 7]
        k += 7
        if last:
            Wm1_ref, bm1_ref, Wm2_ref, bm2_ref = refs[k:k + 4]
            k += 4
        out_ref = refs[k]
        h = (jnp.zeros((RB, H), jnp.float32) if first else h_ref[...])
        for ti, t in enumerate(ts):
            z1t = jnp.concatenate([z1p_ref[0, ti], z1p_ref[1, ti]], axis=1)
            y1t = jnp.concatenate([ytabs[2 * ti][...], ytabs[2 * ti + 1][...]],
                                  axis=1)
            d = dinv_ref[t][:, None]
            emb = jnp.maximum(
                jnp.dot((z1t + y1t) * d, W1_ref[...],
                        preferred_element_type=jnp.float32) + b1_ref[...],
                0.0)
            gi = jnp.dot(emb, Wih_ref[...].T,
                         preferred_element_type=jnp.float32) + bih_ref[...]
            gh = jnp.dot(h, Whh_ref[...].T,
                         preferred_element_type=jnp.float32) + bhh_ref[...]
            r = jax.nn.sigmoid(gi[:, :H] + gh[:, :H])
            z = jax.nn.sigmoid(gi[:, H:2 * H] + gh[:, H:2 * H])
            ng = jnp.tanh(gi[:, 2 * H:] + r * gh[:, 2 * H:])
            h = (1.0 - z) * ng + z * h
        if last:
            h1 = jnp.maximum(
                jnp.dot(h, Wm1_ref[...], preferred_element_type=jnp.float32)
                + bm1_ref[...], 0.0)
            out_ref[...] = jax.nn.sigmoid(
                jnp.dot(h1, Wm2_ref[...], preferred_element_type=jnp.float32)
                + bm2_ref[...])
        else:
            out_ref[...] = h

    return body


def _tc_prep(degp, xcat):
    return pl.pallas_call(
        _prep_block,
        grid=(GRID,),
        in_specs=[pl.BlockSpec((2, T, RB), lambda i: (0, 0, i)),
                  pl.BlockSpec((RB, 12), lambda i: (i, 0))],
        out_specs=[pl.BlockSpec((RB, 16), lambda i: (i, 0)),
                   pl.BlockSpec((T, RB), lambda i: (0, i))],
        out_shape=[jax.ShapeDtypeStruct((NPAD, 16), jnp.float32),
                   jax.ShapeDtypeStruct((T, NPAD), jnp.float32)],
    )(degp, xcat)


def _tc_dense1(z0p, xcat, dinv4, W0, b0, ts):
    nt = len(ts)
    return pl.pallas_call(
        _make_dense1_block(ts),
        grid=(GRID,),
        in_specs=[pl.BlockSpec((2, nt, RB, 16), lambda i: (0, 0, i, 0)),
                  pl.BlockSpec((RB, 12), lambda i: (i, 0)),
                  pl.BlockSpec((T, RB), lambda i: (0, i)),
                  pl.BlockSpec((3, H), lambda i: (0, 0)),
                  pl.BlockSpec((H,), lambda i: (0,))],
        out_specs=[pl.BlockSpec((RB, 32), lambda i: (i, 0))] * (2 * nt),
        out_shape=[jax.ShapeDtypeStruct((NPAD, 32), jnp.float32)] * (2 * nt),
    )(z0p, xcat, dinv4, W0, b0)


def _tc_gru_ab(h, z1p, ys, dinv4, W1, b1, Wih, Whh, bih, bhh,
               Wm1, bm1, Wm2, bm2, ts):
    first = ts[0] == 0
    last = ts[-1] == T - 1
    yspec = [pl.BlockSpec((RB, 32), lambda i: (i, 0))] * (2 * len(ts))
    hspec = [] if first else [pl.BlockSpec((RB, H), lambda i: (i, 0))]
    harg = [] if first else [h]
    wspec = [pl.BlockSpec((T, RB), lambda i: (0, i)),
             pl.BlockSpec((H, H), lambda i: (0, 0)),
             pl.BlockSpec((H,), lambda i: (0,)),
             pl.BlockSpec((3 * H, H), lambda i: (0, 0)),
             pl.BlockSpec((3 * H, H), lambda i: (0, 0)),
             pl.BlockSpec((3 * H,), lambda i: (0,)),
             pl.BlockSpec((3 * H,), lambda i: (0,))]
    wargs = [dinv4, W1, b1, Wih, Whh, bih, bhh]
    if last:
        wspec += [pl.BlockSpec((H, H), lambda i: (0, 0)),
                  pl.BlockSpec((H,), lambda i: (0,)),
                  pl.BlockSpec((H, 1), lambda i: (0, 0)),
                  pl.BlockSpec((1,), lambda i: (0,))]
        wargs += [Wm1, bm1, Wm2, bm2]
        out_w = 1
    else:
        out_w = H
    return pl.pallas_call(
        _make_gru_block(ts, first, last),
        grid=(GRID,),
        in_specs=[pl.BlockSpec((2, len(ts), RB, 32), lambda i: (0, 0, i, 0))]
        + yspec + hspec + wspec,
        out_specs=pl.BlockSpec((RB, out_w), lambda i: (i, 0)),
        out_shape=jax.ShapeDtypeStruct((NPAD, out_w), jnp.float32),
    )(z1p, *ys, *harg, *wargs)


def kernel(node_features_0, node_features_1, node_features_2, node_features_3,
           edges_0, edges_1, edges_2, edges_3,
           W_gcn0, b_gcn0, W_gcn1, b_gcn1, W_ih, W_hh, b_ih, b_hh,
           W_mlp1, b_mlp1, W_mlp2, b_mlp2):
    feats = [node_features_0, node_features_1, node_features_2, node_features_3]
    edges = [edges_0, edges_1, edges_2, edges_3]

    xcat = jnp.concatenate(
        [jnp.pad(x, ((0, NPAD - N), (0, 0))) for x in feats], axis=1)

    pad_src = jnp.arange(E, EPAD, dtype=jnp.int32) % NPAD
    pad_dst = N + jnp.arange(E, EPAD, dtype=jnp.int32) % (NPAD - N)
    srcs, dsts = [], []
    for e in edges:
        srcs.append(jnp.concatenate([e[0], pad_src]))
        dsts.append(jnp.concatenate([e[1], pad_dst]))
    srcf = jnp.stack(srcs).reshape(-1)
    dstf = jnp.stack(dsts).reshape(-1)
    dst3 = dstf.reshape(T, ROWS_E, 128)

    zdeg = jnp.zeros((NPAD,), jnp.float32)
    ones128 = jnp.ones((128,), jnp.float32)
    z16 = jnp.zeros((NPAD, 16), jnp.float32)
    z32 = jnp.zeros((NPAD, 32), jnp.float32)

    degp = _sc_hist(dst3, zdeg, ones128).reshape(2, T, NPAD)
    y0, dinv4 = _tc_prep(degp, xcat)
    z0a = _sc_pass0(y0, srcf, dstf, z16, (0, 1)).reshape(2, 2, NPAD, 16)
    z0b = _sc_pass0(y0, srcf, dstf, z16, (2, 3)).reshape(2, 2, NPAD, 16)
    ysa = _tc_dense1(z0a, xcat, dinv4, W_gcn0, b_gcn0, (0, 1))
    ysb = _tc_dense1(z0b, xcat, dinv4, W_gcn0, b_gcn0, (2, 3))
    z1a = _sc_pass1(ysa, srcf, dstf, z32, (0, 1)).reshape(2, 2, NPAD, 32)
    h01 = _tc_gru_ab(None, z1a, ysa, dinv4, W_gcn1, b_gcn1, W_ih, W_hh,
                     b_ih, b_hh, W_mlp1, b_mlp1, W_mlp2, b_mlp2, (0, 1))
    z1b = _sc_pass1(ysb, srcf, dstf, z32, (2, 3)).reshape(2, 2, NPAD, 32)
    pred = _tc_gru_ab(h01, z1b, ysb, dinv4, W_gcn1, b_gcn1, W_ih, W_hh,
                      b_ih, b_hh, W_mlp1, b_mlp1, W_mlp2, b_mlp2, (2, 3))
    return pred[:N]
